# baseline (speedup 1.0000x reference)
import jax
import jax.numpy as jnp
from jax import lax
from jax.experimental import pallas as pl
from jax.experimental.pallas import tpu as pltpu
from jax.experimental.pallas import tpu_sc as plsc

B = 1024
S = 200
H = 128
N = B * S
LANES = 16
NJ = H // LANES
NUM_WORKERS = 32
PER_W = N // NUM_WORKERS
CHUNK = 128
NCHUNK = PER_W // CHUNK
NSEG = 3
SP = NSEG * S


def _segpos_tc_body(seg_ref, pos_ref, out_ref):
    for g in range(NSEG):
        out_ref[g * S:(g + 1) * S, :] = pos_ref[...] + seg_ref[g:g + 1, :]


def _build_segpos(segment_table, pos_table):
    return pl.pallas_call(
        _segpos_tc_body,
        out_shape=jax.ShapeDtypeStruct((SP, H), jnp.float32),
    )(segment_table, pos_table[:S])


def _sc_body(ids_hbm, sids_hbm, tok_hbm, segpos_hbm, out_hbm,
             idx_all, sidx_all, segpos_v, rows_0, rows_1,
             tsem_0, tsem_1, osem_0, osem_1):
    info = plsc.get_sparse_core_info()
    nc = info.num_cores
    wid = lax.axis_index("s") * nc + lax.axis_index("c")
    wbase = wid * PER_W
    iota = lax.iota(jnp.int32, LANES)

    rows = (rows_0, rows_1)
    tsem = (tsem_0, tsem_1)
    osem = (osem_0, osem_1)

    pltpu.sync_copy(ids_hbm.at[pl.ds(wbase, PER_W)], idx_all)
    pltpu.sync_copy(sids_hbm.at[pl.ds(wbase, PER_W)], sidx_all)
    pltpu.sync_copy(segpos_hbm, segpos_v)

    def fire_tok(ch, s):
        pltpu.async_copy(tok_hbm.at[idx_all.at[pl.ds(ch * CHUNK, CHUNK)]],
                         rows[s], tsem[s])

    def wait_tok(s):
        pltpu.make_async_copy(tok_hbm.at[pl.ds(0, CHUNK)],
                              rows[s], tsem[s]).wait()

    def fire_out(ch, s):
        pltpu.async_copy(rows[s],
                         out_hbm.at[pl.ds(wbase + ch * CHUNK, CHUNK)],
                         osem[s])

    def wait_out(s):
        pltpu.make_async_copy(rows[s], out_hbm.at[pl.ds(0, CHUNK)],
                              osem[s]).wait()

    def add_pass(ch, s):
        def add_group(k, _):
            svec = sidx_all[pl.ds(ch * CHUNK + k * LANES, LANES)]
            gbase = wbase + ch * CHUNK + k * LANES
            for l in range(LANES):
                sval = jnp.sum(jnp.where(iota == l, svec, 0))
                sp = sval * S + lax.rem(gbase + l, S)
                r = k * LANES + l
                for j in range(NJ):
                    plsc.addupdate(rows[s].at[r, pl.ds(LANES * j, LANES)],
                                   segpos_v[sp, pl.ds(LANES * j, LANES)])
            return 0

        lax.fori_loop(0, CHUNK // LANES, add_group, 0)

    fire_tok(0, 0)
    fire_tok(1, 1)
    wait_tok(0)
    add_pass(0, 0)
    fire_out(0, 0)

    def pair_body(c2, _):
        ch = 2 * c2 + 1
        wait_out(0)
        fire_tok(ch + 1, 0)
        wait_tok(1)
        add_pass(ch, 1)
        fire_out(ch, 1)
        wait_out(1)
        fire_tok(ch + 2, 1)
        wait_tok(0)
        add_pass(ch + 1, 0)
        fire_out(ch + 1, 0)
        return 0

    lax.fori_loop(0, (NCHUNK - 2) // 2, pair_body, 0)

    wait_tok(1)
    add_pass(NCHUNK - 1, 1)
    fire_out(NCHUNK - 1, 1)
    wait_out(0)
    wait_out(1)


@jax.jit
def kernel(input_ids, segment_ids, token_table, segment_table, pos_table):
    segpos = _build_segpos(segment_table, pos_table)
    mesh = plsc.VectorSubcoreMesh(core_axis_name="c", subcore_axis_name="s")
    kfn = pl.kernel(
        _sc_body,
        out_type=jax.ShapeDtypeStruct((N, H), jnp.float32),
        mesh=mesh,
        compiler_params=pltpu.CompilerParams(needs_layout_passes=False),
        scratch_types=[
            pltpu.VMEM((PER_W,), jnp.int32),
            pltpu.VMEM((PER_W,), jnp.int32),
            pltpu.VMEM((SP, H), jnp.float32),
            pltpu.VMEM((CHUNK, H), jnp.float32),
            pltpu.VMEM((CHUNK, H), jnp.float32),
            pltpu.SemaphoreType.DMA,
            pltpu.SemaphoreType.DMA,
            pltpu.SemaphoreType.DMA,
            pltpu.SemaphoreType.DMA,
        ],
    )
    out = kfn(input_ids.reshape(N).astype(jnp.int32),
              segment_ids.reshape(N).astype(jnp.int32),
              token_table, segpos)
    return out.reshape(B, S, H)
